# baseline (speedup 1.0000x reference)
import functools

import jax
import jax.numpy as jnp
from jax import lax
from jax.experimental import pallas as pl
from jax.experimental.pallas import tpu as pltpu
from jax.experimental.pallas import tpu_sc as plsc

_NUM_EXAMPLES = 1024
_SEQ_LEN = 50
_VOCAB = 1000

_LANES = 16
_NUM_CORES = 2
_ROWS_PER_W = 2
_VPAD = 1024
_NV = _VPAD // _LANES
_CHUNK = 1024
_NCH = _VPAD // _CHUNK
_ROW_STRIDE = _SEQ_LEN * _VOCAB * _NUM_EXAMPLES // _SEQ_LEN
_G_STRIDE = 16384

_LN2 = 0.6931471805599453
_SQRT2 = 1.4142135623730951


def _vlog(x):
    bits = lax.bitcast_convert_type(x, jnp.int32)
    e = lax.shift_right_logical(bits, 23) - 127
    mbits = lax.bitwise_or(lax.bitwise_and(bits, 0x007FFFFF), 0x3F800000)
    m = lax.bitcast_convert_type(mbits, jnp.float32)
    big = m > _SQRT2
    m = jnp.where(big, m * 0.5, m)
    e = jnp.where(big, e + 1, e)
    t = (m - 1.0) / (m + 1.0)
    t2 = t * t
    p = t * (2.0 + t2 * (2.0 / 3.0 + t2 * (0.4 + t2 * (2.0 / 7.0))))
    return e.astype(jnp.float32) * _LN2 + p


def _xlane(x, op):
    dnums = lax.GatherDimensionNumbers(
        offset_dims=(), collapsed_slice_dims=(0,), start_index_map=(0,))
    for step in (1, 2, 4, 8):
        perm = lax.bitwise_xor(lax.iota(jnp.int32, _LANES), step)
        shuf = lax.gather(x, perm.reshape(_LANES, 1), dnums, (1,),
                          mode=lax.GatherScatterMode.PROMISE_IN_BOUNDS)
        x = op(x, shuf)
    return x


def _sc_body(wf_hbm, base_hbm, out_hbm, idx_v, idxb, rowb, sem):
    wid = lax.axis_index("s") * _NUM_CORES + lax.axis_index("c")
    pltpu.sync_copy(base_hbm, idx_v)
    e = idx_v[...][0]
    row0 = wid * _ROWS_PER_W

    @pl.when(row0 < _SEQ_LEN)
    def _():
        lane = lax.iota(jnp.int32, _LANES)
        lanepat = (lax.shift_right_logical(lane, 3) * 8192
                   + lax.bitwise_and(lane, 7) * 128)
        ebase = (lax.shift_right_logical(e, 7) * 1024
                 + lax.bitwise_and(e, 127))

        ninf = jnp.full((_LANES,), -jnp.inf, jnp.float32)
        copies = []
        for r in range(_ROWS_PER_W):
            rowbase = (row0 + r) * _ROW_STRIDE + ebase
            for g in range(_NV):
                iv = jnp.full((_LANES,), rowbase + g * _G_STRIDE,
                              jnp.int32) + lanepat
                if g == _NV - 2:
                    iv = jnp.where(lane < 8, iv,
                                   jnp.full((_LANES,), rowbase, jnp.int32))
                elif g == _NV - 1:
                    iv = jnp.full((_LANES,), rowbase, jnp.int32)
                idxb[pl.ds(r * _VPAD + g * _LANES, _LANES)] = iv
            for ch in range(_NCH):
                copies.append(pltpu.async_copy(
                    wf_hbm.at[idxb.at[pl.ds(r * _VPAD + ch * _CHUNK,
                                            _CHUNK)]],
                    rowb.at[pl.ds(r * _VPAD + ch * _CHUNK, _CHUNK)],
                    sem))

        for r in range(_ROWS_PER_W):
            b0 = r * _VPAD
            for cp in copies[r * _NCH:(r + 1) * _NCH]:
                cp.wait()
            v = rowb[pl.ds(b0 + 992, _LANES)]
            rowb[pl.ds(b0 + 992, _LANES)] = jnp.where(lane < 8, v, ninf)
            rowb[pl.ds(b0 + 1008, _LANES)] = ninf
            m = rowb[pl.ds(b0, _LANES)]
            for k in range(1, _NV):
                m = jnp.maximum(m, rowb[pl.ds(b0 + k * _LANES, _LANES)])
            mb = _xlane(m, jnp.maximum)
            s = jnp.zeros((_LANES,), jnp.float32)
            for k in range(_NV):
                s = s + jnp.exp(rowb[pl.ds(b0 + k * _LANES, _LANES)] - mb)
            off = mb + _vlog(_xlane(s, jnp.add))
            for k in range(_NV):
                rowb[pl.ds(b0 + k * _LANES, _LANES)] = (
                    rowb[pl.ds(b0 + k * _LANES, _LANES)] - off)
            pltpu.sync_copy(
                rowb.at[pl.ds(b0, _VOCAB)],
                out_hbm.at[pl.ds((row0 + r) * _VOCAB, _VOCAB)])


_sc_kernel = functools.partial(
    pl.kernel,
    mesh=plsc.VectorSubcoreMesh(core_axis_name="c", subcore_axis_name="s"),
    out_type=jax.ShapeDtypeStruct((_SEQ_LEN * _VOCAB,), jnp.float32),
    scratch_types=[
        pltpu.VMEM((_LANES,), jnp.int32),
        pltpu.VMEM((_ROWS_PER_W * _VPAD,), jnp.int32),
        pltpu.VMEM((_ROWS_PER_W * _VPAD,), jnp.float32),
        pltpu.SemaphoreType.DMA,
    ],
)(_sc_body)


def kernel(weights, example_idx):
    wf = (weights.transpose(1, 2, 0)
          .reshape(_SEQ_LEN * _VOCAB // 8, 8, _NUM_EXAMPLES // 128, 128)
          .swapaxes(1, 2)
          .reshape(_SEQ_LEN * _VOCAB * _NUM_EXAMPLES))
    base = jnp.broadcast_to(jnp.asarray(example_idx, jnp.int32), (_LANES,))
    return _sc_kernel(wf, base).reshape(_SEQ_LEN, _VOCAB)


# speedup vs baseline: 1.0330x; 1.0330x over previous
import functools

import jax
import jax.numpy as jnp
from jax import lax
from jax.experimental import pallas as pl
from jax.experimental.pallas import tpu as pltpu
from jax.experimental.pallas import tpu_sc as plsc

_NUM_EXAMPLES = 1024
_SEQ_LEN = 50
_VOCAB = 1000

_LANES = 16
_NUM_CORES = 2
_ROWS_PER_W = 2
_VPAD = 1024
_NV = _VPAD // _LANES
_CHUNK = 1024
_NCH = _VPAD // _CHUNK
_ROW_STRIDE = _SEQ_LEN * _VOCAB * _NUM_EXAMPLES // _SEQ_LEN
_G_STRIDE = 16384

_LN2 = 0.6931471805599453
_SQRT2 = 1.4142135623730951


def _vlog(x):
    bits = lax.bitcast_convert_type(x, jnp.int32)
    e = lax.shift_right_logical(bits, 23) - 127
    mbits = lax.bitwise_or(lax.bitwise_and(bits, 0x007FFFFF), 0x3F800000)
    m = lax.bitcast_convert_type(mbits, jnp.float32)
    big = m > _SQRT2
    m = jnp.where(big, m * 0.5, m)
    e = jnp.where(big, e + 1, e)
    t = (m - 1.0) / (m + 1.0)
    t2 = t * t
    p = t * (2.0 + t2 * (2.0 / 3.0 + t2 * (0.4 + t2 * (2.0 / 7.0))))
    return e.astype(jnp.float32) * _LN2 + p


def _xlane(x, op):
    dnums = lax.GatherDimensionNumbers(
        offset_dims=(), collapsed_slice_dims=(0,), start_index_map=(0,))
    for step in (1, 2, 4, 8):
        perm = lax.bitwise_xor(lax.iota(jnp.int32, _LANES), step)
        shuf = lax.gather(x, perm.reshape(_LANES, 1), dnums, (1,),
                          mode=lax.GatherScatterMode.PROMISE_IN_BOUNDS)
        x = op(x, shuf)
    return x


def _sc_body(wf_hbm, base_hbm, out_hbm, idx_v, idxb, rowb, sem):
    wid = lax.axis_index("s") * _NUM_CORES + lax.axis_index("c")
    pltpu.sync_copy(base_hbm, idx_v)
    e = idx_v[...][0]
    row0 = wid * _ROWS_PER_W

    @pl.when(row0 < _SEQ_LEN)
    def _():
        lane = lax.iota(jnp.int32, _LANES)
        lanepat = (lax.shift_right_logical(lane, 3) * 8192
                   + lax.bitwise_and(lane, 7) * 128)
        ebase = (lax.shift_right_logical(e, 7) * 1024
                 + lax.bitwise_and(e, 127))

        ninf = jnp.full((_LANES,), -jnp.inf, jnp.float32)
        copies = []
        for r in range(_ROWS_PER_W):
            rowbase = (row0 + r) * _ROW_STRIDE + ebase
            def _ibody(g, _):
                idxb[pl.ds(r * _VPAD + g * _LANES, _LANES)] = (
                    jnp.full((_LANES,), rowbase + g * _G_STRIDE,
                             jnp.int32) + lanepat)
                return 0
            lax.fori_loop(0, _NV - 2, _ibody, 0, unroll=8)
            iv = jnp.full((_LANES,), rowbase + (_NV - 2) * _G_STRIDE,
                          jnp.int32) + lanepat
            iv = jnp.where(lane < 8, iv,
                           jnp.full((_LANES,), rowbase, jnp.int32))
            idxb[pl.ds(r * _VPAD + (_NV - 2) * _LANES, _LANES)] = iv
            idxb[pl.ds(r * _VPAD + (_NV - 1) * _LANES, _LANES)] = (
                jnp.full((_LANES,), rowbase, jnp.int32))
            for ch in range(_NCH):
                copies.append(pltpu.async_copy(
                    wf_hbm.at[idxb.at[pl.ds(r * _VPAD + ch * _CHUNK,
                                            _CHUNK)]],
                    rowb.at[pl.ds(r * _VPAD + ch * _CHUNK, _CHUNK)],
                    sem))

        for r in range(_ROWS_PER_W):
            b0 = r * _VPAD
            for cp in copies[r * _NCH:(r + 1) * _NCH]:
                cp.wait()
            v = rowb[pl.ds(b0 + 992, _LANES)]
            rowb[pl.ds(b0 + 992, _LANES)] = jnp.where(lane < 8, v, ninf)
            rowb[pl.ds(b0 + 1008, _LANES)] = ninf
            m = lax.fori_loop(
                0, _NV,
                lambda k, m: jnp.maximum(
                    m, rowb[pl.ds(b0 + k * _LANES, _LANES)]),
                ninf, unroll=8)
            mb = _xlane(m, jnp.maximum)
            s = lax.fori_loop(
                0, _NV,
                lambda k, s: s + jnp.exp(
                    rowb[pl.ds(b0 + k * _LANES, _LANES)] - mb),
                jnp.zeros((_LANES,), jnp.float32), unroll=8)
            off = mb + _vlog(_xlane(s, jnp.add))

            def _obody(k, _):
                rowb[pl.ds(b0 + k * _LANES, _LANES)] = (
                    rowb[pl.ds(b0 + k * _LANES, _LANES)] - off)
                return 0
            lax.fori_loop(0, _NV, _obody, 0, unroll=8)
            pltpu.sync_copy(
                rowb.at[pl.ds(b0, _VOCAB)],
                out_hbm.at[pl.ds((row0 + r) * _VOCAB, _VOCAB)])


_sc_kernel = functools.partial(
    pl.kernel,
    mesh=plsc.VectorSubcoreMesh(core_axis_name="c", subcore_axis_name="s"),
    out_type=jax.ShapeDtypeStruct((_SEQ_LEN * _VOCAB,), jnp.float32),
    scratch_types=[
        pltpu.VMEM((_LANES,), jnp.int32),
        pltpu.VMEM((_ROWS_PER_W * _VPAD,), jnp.int32),
        pltpu.VMEM((_ROWS_PER_W * _VPAD,), jnp.float32),
        pltpu.SemaphoreType.DMA,
    ],
)(_sc_body)


def kernel(weights, example_idx):
    wf = (weights.transpose(1, 2, 0)
          .reshape(_SEQ_LEN * _VOCAB // 8, 8, _NUM_EXAMPLES // 128, 128)
          .swapaxes(1, 2)
          .reshape(_SEQ_LEN * _VOCAB * _NUM_EXAMPLES))
    base = jnp.broadcast_to(jnp.asarray(example_idx, jnp.int32), (_LANES,))
    return _sc_kernel(wf, base).reshape(_SEQ_LEN, _VOCAB)
